# Initial kernel scaffold; baseline (speedup 1.0000x reference)
import jax
import jax.numpy as jnp
from jax.experimental import pallas as pl


def kernel(points, params):
    raise NotImplementedError("write your pallas kernel here")


# speedup vs baseline: 1.0961x; 1.0961x over previous
import functools

import jax
import jax.numpy as jnp
import numpy as np
from jax.experimental import pallas as pl

_NPOINTS = [2048, 512, 128, 32]
_RADIUS = [[0.1, 0.5], [0.5, 1.0], [1.0, 2.0], [2.0, 4.0]]
_NSAMPLE = [[16, 32], [16, 32], [16, 32], [16, 32]]
_BN_EPS = 1e-5


def _fold_bn(params):
    out = []
    for (W, b, gamma, beta) in params:
        s = gamma / np.sqrt(1.0 + _BN_EPS)
        out.append((W * s[None, :], b * s + beta))
    return out


def _mlp_pallas(h, wbs):
    M, Cin = h.shape
    BM = min(512, max(8, M))
    Mp = pl.cdiv(M, BM) * BM
    if Mp != M:
        h = jnp.pad(h, ((0, Mp - M), (0, 0)))
    n = len(wbs)
    outC = wbs[-1][0].shape[1]

    def body(*refs):
        x = refs[0][...]
        for i in range(n):
            W = refs[1 + 2 * i][...]
            b = refs[2 + 2 * i][...]
            x = jnp.maximum(
                jnp.dot(x, W, preferred_element_type=jnp.float32) + b, 0.0)
        refs[-1][...] = x

    in_specs = [pl.BlockSpec((BM, Cin), lambda i: (i, 0))]
    args = [h]
    for (W, bvec) in wbs:
        in_specs.append(pl.BlockSpec(W.shape, lambda i: (0, 0)))
        in_specs.append(pl.BlockSpec((1, W.shape[1]), lambda i: (0, 0)))
        args += [W, bvec.reshape(1, -1)]
    out = pl.pallas_call(
        body,
        grid=(Mp // BM,),
        in_specs=in_specs,
        out_specs=pl.BlockSpec((BM, outC), lambda i: (i, 0)),
        out_shape=jax.ShapeDtypeStruct((Mp, outC), jnp.float32),
    )(*args)
    return out[:M]


def _pairwise_d2(a, b):
    aa = jnp.sum(a * a, axis=-1)[:, :, None]
    bb = jnp.sum(b * b, axis=-1)[:, None, :]
    ab = jnp.einsum('bnd,bmd->bnm', a, b)
    return jnp.maximum(aa + bb - 2.0 * ab, 0.0)


def _fps(xyz, npoint):
    b, n, _ = xyz.shape

    def body(i, state):
        idxs, dists, far = state
        idxs = idxs.at[:, i].set(far)
        centroid = jnp.take_along_axis(xyz, far[:, None, None], axis=1)
        d = jnp.sum((xyz - centroid) ** 2, axis=-1)
        dists = jnp.minimum(dists, d)
        far = jnp.argmax(dists, axis=-1).astype(jnp.int32)
        return (idxs, dists, far)

    idxs = jnp.zeros((b, npoint), jnp.int32)
    dists = jnp.full((b, n), 1e10, jnp.float32)
    far = jnp.zeros((b,), jnp.int32)
    idxs, _, _ = jax.lax.fori_loop(0, npoint, body, (idxs, dists, far))
    return idxs


def _ball_query(radius, nsample, d2):
    n = d2.shape[-1]
    mask = d2 < radius * radius
    rank = jnp.cumsum(mask.astype(jnp.int32), axis=-1) - 1
    rank = jnp.where(mask, rank, n)
    sel = rank < nsample
    onehot = (rank[..., None] == jnp.arange(nsample, dtype=jnp.int32))
    jidx = jnp.arange(n, dtype=jnp.int32)[None, None, :, None]
    slot = jnp.sum(jnp.where(onehot, jidx + 1, 0), axis=2) - 1
    first = slot[..., :1]
    first = jnp.where(first < 0, 0, first)
    return jnp.where(slot < 0, first, slot)


def _batched_gather(x, idx):
    return jax.vmap(lambda a, i: a[i])(x, idx)


def _sa_msg(xyz, feats, npoint, radii, nsamples, scale_params):
    fps_idx = _fps(xyz, npoint)
    new_xyz = _batched_gather(xyz, fps_idx)
    d2 = _pairwise_d2(new_xyz, xyz)
    outs = []
    for radius, nsample, params in zip(radii, nsamples, scale_params):
        idx = _ball_query(radius, nsample, d2)
        grouped_xyz = _batched_gather(xyz, idx) - new_xyz[:, :, None, :]
        if feats is not None:
            grouped_feats = _batched_gather(feats, idx)
            g = jnp.concatenate([grouped_xyz, grouped_feats], axis=-1)
        else:
            g = grouped_xyz
        bq, nq, ns, ci = g.shape
        h = _mlp_pallas(g.reshape(bq * nq * ns, ci), _fold_bn(params))
        h = h.reshape(bq, nq, ns, -1)
        outs.append(jnp.max(h, axis=2))
    return new_xyz, jnp.concatenate(outs, axis=-1)


def _fp(xyz1, xyz2, feats1, feats2, params):
    d2 = _pairwise_d2(xyz1, xyz2)
    neg_vals, idx = jax.lax.top_k(-d2, 3)
    dist = jnp.sqrt(jnp.maximum(-neg_vals, 0.0))
    dist_recip = 1.0 / (dist + 1e-8)
    norm = jnp.sum(dist_recip, axis=2, keepdims=True)
    weight = dist_recip / norm
    gathered = _batched_gather(feats2, idx)
    interp = jnp.sum(gathered * weight[..., None], axis=2)
    h = jnp.concatenate([interp, feats1], axis=-1) if feats1 is not None else interp
    bq, nq, ci = h.shape
    out = _mlp_pallas(h.reshape(bq * nq, ci), _fold_bn(params))
    return out.reshape(bq, nq, -1)


def kernel(points, params):
    xyz = points[:, :, :3]
    feats = points[:, :, 3:] if points.shape[2] > 3 else None
    l_xyz = [xyz]
    l_feats = [feats]
    for i in range(len(_NPOINTS)):
        nx, nf = _sa_msg(l_xyz[i], l_feats[i], _NPOINTS[i], _RADIUS[i],
                         _NSAMPLE[i], params["sa"][i])
        l_xyz.append(nx)
        l_feats.append(nf)
    for i in range(-1, -(len(_RADIUS) + 1), -1):
        l_feats[i - 1] = _fp(l_xyz[i - 1], l_xyz[i], l_feats[i - 1],
                             l_feats[i], params["fp"][i])
    return tuple(jnp.transpose(f, (0, 2, 1)) for f in l_feats)


# speedup vs baseline: 1.8047x; 1.6464x over previous
import functools

import jax
import jax.numpy as jnp
import numpy as np
from jax.experimental import pallas as pl

_NPOINTS = [2048, 512, 128, 32]
_RADIUS = [[0.1, 0.5], [0.5, 1.0], [1.0, 2.0], [2.0, 4.0]]
_NSAMPLE = [[16, 32], [16, 32], [16, 32], [16, 32]]
_BN_EPS = 1e-5


def _fold_bn(params):
    out = []
    for (W, b, gamma, beta) in params:
        s = gamma / np.sqrt(1.0 + _BN_EPS)
        out.append((W * s[None, :], b * s + beta))
    return out


def _mlp_pallas(h, wbs):
    M, Cin = h.shape
    BM = min(512, max(8, M))
    Mp = pl.cdiv(M, BM) * BM
    if Mp != M:
        h = jnp.pad(h, ((0, Mp - M), (0, 0)))
    n = len(wbs)
    outC = wbs[-1][0].shape[1]

    def body(*refs):
        x = refs[0][...]
        for i in range(n):
            W = refs[1 + 2 * i][...]
            b = refs[2 + 2 * i][...]
            x = jnp.maximum(
                jnp.dot(x, W, preferred_element_type=jnp.float32) + b, 0.0)
        refs[-1][...] = x

    in_specs = [pl.BlockSpec((BM, Cin), lambda i: (i, 0))]
    args = [h]
    for (W, bvec) in wbs:
        in_specs.append(pl.BlockSpec(W.shape, lambda i: (0, 0)))
        in_specs.append(pl.BlockSpec((1, W.shape[1]), lambda i: (0, 0)))
        args += [W, bvec.reshape(1, -1)]
    out = pl.pallas_call(
        body,
        grid=(Mp // BM,),
        in_specs=in_specs,
        out_specs=pl.BlockSpec((BM, outC), lambda i: (i, 0)),
        out_shape=jax.ShapeDtypeStruct((Mp, outC), jnp.float32),
    )(*args)
    return out[:M]


def _pairwise_d2(a, b):
    aa = jnp.sum(a * a, axis=-1)[:, :, None]
    bb = jnp.sum(b * b, axis=-1)[:, None, :]
    ab = jnp.einsum('bnd,bmd->bnm', a, b)
    return jnp.maximum(aa + bb - 2.0 * ab, 0.0)


def _fps_new_xyz(xyz, npoint):
    B_, N, _ = xyz.shape
    Nr = N // 128
    planes = jnp.transpose(xyz, (2, 0, 1)).reshape(3, B_, Nr, 128)

    def body(p_ref, ox_ref, oy_ref, oz_ref):
        row_io = jax.lax.broadcasted_iota(jnp.int32, (Nr, 128), 0)
        col_io = jax.lax.broadcasted_iota(jnp.int32, (Nr, 128), 1)
        flat = row_io * 128 + col_io
        xs = [p_ref[0, b] for b in range(B_)]
        ys = [p_ref[1, b] for b in range(B_)]
        zs = [p_ref[2, b] for b in range(B_)]

        def step(i, carry):
            fars, dists = carry
            new_fars, new_dists = [], []
            for b in range(B_):
                x, y, z = xs[b], ys[b], zs[b]
                sel = flat == fars[b]
                cx = jnp.sum(jnp.where(sel, x, 0.0))
                cy = jnp.sum(jnp.where(sel, y, 0.0))
                cz = jnp.sum(jnp.where(sel, z, 0.0))
                ox_ref[pl.ds(i, 1), b] = jnp.full((1,), cx, jnp.float32)
                oy_ref[pl.ds(i, 1), b] = jnp.full((1,), cy, jnp.float32)
                oz_ref[pl.ds(i, 1), b] = jnp.full((1,), cz, jnp.float32)
                dx, dy, dz = x - cx, y - cy, z - cz
                d = dx * dx + dy * dy + dz * dz
                nd = jnp.minimum(dists[b], d)
                m = jnp.max(nd)
                nf = jnp.min(jnp.where(nd == m, flat, jnp.int32(N)))
                new_fars.append(nf)
                new_dists.append(nd)
            return (tuple(new_fars), tuple(new_dists))

        fars0 = tuple(jnp.int32(0) for _ in range(B_))
        dists0 = tuple(jnp.full((Nr, 128), 1e10, jnp.float32)
                       for _ in range(B_))
        jax.lax.fori_loop(0, npoint, step, (fars0, dists0))

    ox, oy, oz = pl.pallas_call(
        body,
        out_shape=[jax.ShapeDtypeStruct((npoint, B_), jnp.float32)] * 3,
    )(planes)
    return jnp.stack([ox, oy, oz], axis=-1).transpose(1, 0, 2)


def _ball_query(radius, nsample, d2):
    n = d2.shape[-1]
    mask = d2 < radius * radius
    rank = jnp.cumsum(mask.astype(jnp.int32), axis=-1) - 1
    rank = jnp.where(mask, rank, n)
    sel = rank < nsample
    onehot = (rank[..., None] == jnp.arange(nsample, dtype=jnp.int32))
    jidx = jnp.arange(n, dtype=jnp.int32)[None, None, :, None]
    slot = jnp.sum(jnp.where(onehot, jidx + 1, 0), axis=2) - 1
    first = slot[..., :1]
    first = jnp.where(first < 0, 0, first)
    return jnp.where(slot < 0, first, slot)


def _batched_gather(x, idx):
    return jax.vmap(lambda a, i: a[i])(x, idx)


def _sa_msg(xyz, feats, npoint, radii, nsamples, scale_params):
    new_xyz = _fps_new_xyz(xyz, npoint)
    d2 = _pairwise_d2(new_xyz, xyz)
    outs = []
    for radius, nsample, params in zip(radii, nsamples, scale_params):
        idx = _ball_query(radius, nsample, d2)
        grouped_xyz = _batched_gather(xyz, idx) - new_xyz[:, :, None, :]
        if feats is not None:
            grouped_feats = _batched_gather(feats, idx)
            g = jnp.concatenate([grouped_xyz, grouped_feats], axis=-1)
        else:
            g = grouped_xyz
        bq, nq, ns, ci = g.shape
        h = _mlp_pallas(g.reshape(bq * nq * ns, ci), _fold_bn(params))
        h = h.reshape(bq, nq, ns, -1)
        outs.append(jnp.max(h, axis=2))
    return new_xyz, jnp.concatenate(outs, axis=-1)


def _fp(xyz1, xyz2, feats1, feats2, params):
    d2 = _pairwise_d2(xyz1, xyz2)
    neg_vals, idx = jax.lax.top_k(-d2, 3)
    dist = jnp.sqrt(jnp.maximum(-neg_vals, 0.0))
    dist_recip = 1.0 / (dist + 1e-8)
    norm = jnp.sum(dist_recip, axis=2, keepdims=True)
    weight = dist_recip / norm
    gathered = _batched_gather(feats2, idx)
    interp = jnp.sum(gathered * weight[..., None], axis=2)
    h = jnp.concatenate([interp, feats1], axis=-1) if feats1 is not None else interp
    bq, nq, ci = h.shape
    out = _mlp_pallas(h.reshape(bq * nq, ci), _fold_bn(params))
    return out.reshape(bq, nq, -1)


def kernel(points, params):
    xyz = points[:, :, :3]
    feats = points[:, :, 3:] if points.shape[2] > 3 else None
    l_xyz = [xyz]
    l_feats = [feats]
    for i in range(len(_NPOINTS)):
        nx, nf = _sa_msg(l_xyz[i], l_feats[i], _NPOINTS[i], _RADIUS[i],
                         _NSAMPLE[i], params["sa"][i])
        l_xyz.append(nx)
        l_feats.append(nf)
    for i in range(-1, -(len(_RADIUS) + 1), -1):
        l_feats[i - 1] = _fp(l_xyz[i - 1], l_xyz[i], l_feats[i - 1],
                             l_feats[i], params["fp"][i])
    return tuple(jnp.transpose(f, (0, 2, 1)) for f in l_feats)


# speedup vs baseline: 3.7982x; 2.1046x over previous
import functools

import jax
import jax.numpy as jnp
import numpy as np
from jax import lax
from jax.experimental import pallas as pl
from jax.experimental.pallas import tpu as pltpu
from jax.experimental.pallas import tpu_sc as plsc

_NPOINTS = [2048, 512, 128, 32]
_RADIUS = [[0.1, 0.5], [0.5, 1.0], [1.0, 2.0], [2.0, 4.0]]
_NSAMPLE = [[16, 32], [16, 32], [16, 32], [16, 32]]
_BN_EPS = 1e-5


def _fold_bn(params):
    out = []
    for (W, b, gamma, beta) in params:
        s = gamma / np.sqrt(1.0 + _BN_EPS)
        out.append((W * s[None, :], b * s + beta))
    return out


def _mlp_pallas(h, wbs):
    M, Cin = h.shape
    BM = min(512, max(8, M))
    Mp = pl.cdiv(M, BM) * BM
    if Mp != M:
        h = jnp.pad(h, ((0, Mp - M), (0, 0)))
    n = len(wbs)
    outC = wbs[-1][0].shape[1]

    def body(*refs):
        x = refs[0][...]
        for i in range(n):
            W = refs[1 + 2 * i][...]
            b = refs[2 + 2 * i][...]
            x = jnp.maximum(
                jnp.dot(x, W, preferred_element_type=jnp.float32) + b, 0.0)
        refs[-1][...] = x

    in_specs = [pl.BlockSpec((BM, Cin), lambda i: (i, 0))]
    args = [h]
    for (W, bvec) in wbs:
        in_specs.append(pl.BlockSpec(W.shape, lambda i: (0, 0)))
        in_specs.append(pl.BlockSpec((1, W.shape[1]), lambda i: (0, 0)))
        args += [W, bvec.reshape(1, -1)]
    out = pl.pallas_call(
        body,
        grid=(Mp // BM,),
        in_specs=in_specs,
        out_specs=pl.BlockSpec((BM, outC), lambda i: (i, 0)),
        out_shape=jax.ShapeDtypeStruct((Mp, outC), jnp.float32),
    )(*args)
    return out[:M]


_SC_NC = 2
_SC_NS = 16


def _sc_gather_rows(table, idx):
    V, D = table.shape
    M = idx.shape[0]
    NW = _SC_NC * _SC_NS
    mpw = M // NW
    ch = min(128, mpw, max(8, (32768 // D) // 8 * 8))
    while mpw % ch:
        ch -= 8
    nch = mpw // ch
    mesh = plsc.VectorSubcoreMesh(core_axis_name="c", subcore_axis_name="s")

    @functools.partial(
        pl.kernel, mesh=mesh,
        compiler_params=pltpu.CompilerParams(use_tc_tiling_on_sc=False),
        out_type=jax.ShapeDtypeStruct((M, D), jnp.float32),
        scratch_types=[
            pltpu.VMEM((mpw,), jnp.int32),
            pltpu.VMEM((ch, D), jnp.float32),
            pltpu.SemaphoreType.DMA,
        ],
    )
    def k(table_hbm, idx_hbm, out_hbm, idx_v, rows_v, sem):
        wid = lax.axis_index("s") * _SC_NC + lax.axis_index("c")
        base = wid * mpw
        pltpu.sync_copy(idx_hbm.at[pl.ds(base, mpw)], idx_v)

        def chunk(ci, _):
            off = ci * ch
            pltpu.async_copy(
                table_hbm.at[idx_v.at[pl.ds(off, ch)]], rows_v, sem).wait()
            pltpu.sync_copy(rows_v, out_hbm.at[pl.ds(base + off, ch)])
            return 0

        lax.fori_loop(0, nch, chunk, 0)

    return k(table, idx)


def _pad_lanes(x, mult=16):
    c = x.shape[-1]
    p = (-c) % mult
    if p:
        x = jnp.pad(x, [(0, 0)] * (x.ndim - 1) + [(0, p)])
    return x


def _pairwise_d2(a, b):
    aa = jnp.sum(a * a, axis=-1)[:, :, None]
    bb = jnp.sum(b * b, axis=-1)[:, None, :]
    ab = jnp.einsum('bnd,bmd->bnm', a, b)
    return jnp.maximum(aa + bb - 2.0 * ab, 0.0)


def _fps_new_xyz(xyz, npoint):
    B_, N, _ = xyz.shape
    Nr = N // 128
    planes = jnp.transpose(xyz, (2, 0, 1)).reshape(3, B_, Nr, 128)

    def body(p_ref, ox_ref, oy_ref, oz_ref):
        row_io = jax.lax.broadcasted_iota(jnp.int32, (Nr, 128), 0)
        col_io = jax.lax.broadcasted_iota(jnp.int32, (Nr, 128), 1)
        flat = row_io * 128 + col_io
        xs = [p_ref[0, b] for b in range(B_)]
        ys = [p_ref[1, b] for b in range(B_)]
        zs = [p_ref[2, b] for b in range(B_)]

        def step(i, carry):
            fars, dists = carry
            new_fars, new_dists = [], []
            for b in range(B_):
                x, y, z = xs[b], ys[b], zs[b]
                sel = flat == fars[b]
                cx = jnp.sum(jnp.where(sel, x, 0.0))
                cy = jnp.sum(jnp.where(sel, y, 0.0))
                cz = jnp.sum(jnp.where(sel, z, 0.0))
                ox_ref[pl.ds(i, 1), b] = jnp.full((1,), cx, jnp.float32)
                oy_ref[pl.ds(i, 1), b] = jnp.full((1,), cy, jnp.float32)
                oz_ref[pl.ds(i, 1), b] = jnp.full((1,), cz, jnp.float32)
                dx, dy, dz = x - cx, y - cy, z - cz
                d = dx * dx + dy * dy + dz * dz
                nd = jnp.minimum(dists[b], d)
                m = jnp.max(nd)
                nf = jnp.min(jnp.where(nd == m, flat, jnp.int32(N)))
                new_fars.append(nf)
                new_dists.append(nd)
            return (tuple(new_fars), tuple(new_dists))

        fars0 = tuple(jnp.int32(0) for _ in range(B_))
        dists0 = tuple(jnp.full((Nr, 128), 1e10, jnp.float32)
                       for _ in range(B_))
        jax.lax.fori_loop(0, npoint, step, (fars0, dists0))

    ox, oy, oz = pl.pallas_call(
        body,
        out_shape=[jax.ShapeDtypeStruct((npoint, B_), jnp.float32)] * 3,
    )(planes)
    return jnp.stack([ox, oy, oz], axis=-1).transpose(1, 0, 2)


def _ball_query(radius, nsample, d2):
    n = d2.shape[-1]
    mask = d2 < radius * radius
    rank = jnp.cumsum(mask.astype(jnp.int32), axis=-1) - 1
    rank = jnp.where(mask, rank, n)
    sel = rank < nsample
    onehot = (rank[..., None] == jnp.arange(nsample, dtype=jnp.int32))
    jidx = jnp.arange(n, dtype=jnp.int32)[None, None, :, None]
    slot = jnp.sum(jnp.where(onehot, jidx + 1, 0), axis=2) - 1
    first = slot[..., :1]
    first = jnp.where(first < 0, 0, first)
    return jnp.where(slot < 0, first, slot)


def _batched_gather(x, idx):
    return jax.vmap(lambda a, i: a[i])(x, idx)


def _sa_msg(xyz, feats, npoint, radii, nsamples, scale_params):
    B_, N, _ = xyz.shape
    new_xyz = _fps_new_xyz(xyz, npoint)
    d2 = _pairwise_d2(new_xyz, xyz)
    table = _pad_lanes(jnp.concatenate([xyz, feats], axis=-1)
                       if feats is not None else xyz)
    tflat = table.reshape(B_ * N, table.shape[-1])
    boff = (jnp.arange(B_, dtype=jnp.int32) * N)[:, None, None]
    outs = []
    for radius, nsample, params in zip(radii, nsamples, scale_params):
        idx = _ball_query(radius, nsample, d2)
        rows = _sc_gather_rows(tflat, (idx + boff).reshape(-1))
        rows = rows.reshape(B_, npoint, nsample, -1)
        grouped_xyz = rows[..., :3] - new_xyz[:, :, None, :]
        ci = 3 + (feats.shape[-1] if feats is not None else 0)
        g = jnp.concatenate([grouped_xyz, rows[..., 3:ci]], axis=-1)
        h = _mlp_pallas(g.reshape(B_ * npoint * nsample, ci),
                        _fold_bn(params))
        h = h.reshape(B_, npoint, nsample, -1)
        outs.append(jnp.max(h, axis=2))
    return new_xyz, jnp.concatenate(outs, axis=-1)


def _three_nn_pallas(xyz1, xyz2):
    B_, n1, _ = xyz1.shape
    n2 = xyz2.shape[1]
    BQ = min(512, n1)
    x2t = jnp.transpose(xyz2, (0, 2, 1))

    def body(a_ref, bt_ref, idx_ref, w_ref):
        a = a_ref[0]
        bt = bt_ref[0]
        aa = jnp.sum(a * a, axis=1, keepdims=True)
        bb = jnp.sum(bt * bt, axis=0, keepdims=True)
        ab = jnp.dot(a, bt, preferred_element_type=jnp.float32)
        d2 = jnp.maximum(aa + bb - 2.0 * ab, 0.0)
        col = jax.lax.broadcasted_iota(jnp.int32, (BQ, n2), 1)
        dks = []
        for k in range(3):
            m = jnp.min(d2, axis=1, keepdims=True)
            am = jnp.min(jnp.where(d2 == m, col, n2), axis=1, keepdims=True)
            idx_ref[0, :, k] = am[:, 0]
            dks.append(m)
            d2 = jnp.where(col == am, jnp.float32(jnp.inf), d2)
        dist = jnp.sqrt(jnp.concatenate(dks, axis=1))
        recip = 1.0 / (dist + 1e-8)
        w_ref[0] = recip / jnp.sum(recip, axis=1, keepdims=True)

    idx, w = pl.pallas_call(
        body,
        grid=(B_, n1 // BQ),
        in_specs=[
            pl.BlockSpec((1, BQ, 3), lambda b, i: (b, i, 0)),
            pl.BlockSpec((1, 3, n2), lambda b, i: (b, 0, 0)),
        ],
        out_specs=[
            pl.BlockSpec((1, BQ, 3), lambda b, i: (b, i, 0)),
            pl.BlockSpec((1, BQ, 3), lambda b, i: (b, i, 0)),
        ],
        out_shape=[
            jax.ShapeDtypeStruct((B_, n1, 3), jnp.int32),
            jax.ShapeDtypeStruct((B_, n1, 3), jnp.float32),
        ],
    )(xyz1, x2t)
    return idx, w


def _fp(xyz1, xyz2, feats1, feats2, params):
    B_, n1 = xyz1.shape[0], xyz1.shape[1]
    n2, C2 = feats2.shape[1], feats2.shape[2]
    idx, weight = _three_nn_pallas(xyz1, xyz2)
    boff = (jnp.arange(B_, dtype=jnp.int32) * n2)[:, None, None]
    rows = _sc_gather_rows(feats2.reshape(B_ * n2, C2),
                           (idx + boff).reshape(-1))
    gathered = rows.reshape(B_, n1, 3, C2)
    interp = jnp.sum(gathered * weight[..., None], axis=2)
    h = jnp.concatenate([interp, feats1], axis=-1) if feats1 is not None else interp
    bq, nq, ci = h.shape
    out = _mlp_pallas(h.reshape(bq * nq, ci), _fold_bn(params))
    return out.reshape(bq, nq, -1)


def kernel(points, params):
    xyz = points[:, :, :3]
    feats = points[:, :, 3:] if points.shape[2] > 3 else None
    l_xyz = [xyz]
    l_feats = [feats]
    for i in range(len(_NPOINTS)):
        nx, nf = _sa_msg(l_xyz[i], l_feats[i], _NPOINTS[i], _RADIUS[i],
                         _NSAMPLE[i], params["sa"][i])
        l_xyz.append(nx)
        l_feats.append(nf)
    for i in range(-1, -(len(_RADIUS) + 1), -1):
        l_feats[i - 1] = _fp(l_xyz[i - 1], l_xyz[i], l_feats[i - 1],
                             l_feats[i], params["fp"][i])
    return tuple(jnp.transpose(f, (0, 2, 1)) for f in l_feats)


# speedup vs baseline: 14.3258x; 3.7718x over previous
import functools

import jax
import jax.numpy as jnp
import numpy as np
from jax import lax
from jax.experimental import pallas as pl
from jax.experimental.pallas import tpu as pltpu
from jax.experimental.pallas import tpu_sc as plsc

_NPOINTS = [2048, 512, 128, 32]
_RADIUS = [[0.1, 0.5], [0.5, 1.0], [1.0, 2.0], [2.0, 4.0]]
_NSAMPLE = [[16, 32], [16, 32], [16, 32], [16, 32]]
_BN_EPS = 1e-5


def _fold_bn(params):
    out = []
    for (W, b, gamma, beta) in params:
        s = gamma / np.sqrt(1.0 + _BN_EPS)
        out.append((W * s[None, :], b * s + beta))
    return out


def _mlp_pallas(h, wbs):
    M, Cin = h.shape
    BM = min(512, max(8, M))
    Mp = pl.cdiv(M, BM) * BM
    if Mp != M:
        h = jnp.pad(h, ((0, Mp - M), (0, 0)))
    n = len(wbs)
    outC = wbs[-1][0].shape[1]

    def body(*refs):
        x = refs[0][...]
        for i in range(n):
            W = refs[1 + 2 * i][...]
            b = refs[2 + 2 * i][...]
            x = jnp.maximum(
                jnp.dot(x, W, preferred_element_type=jnp.float32) + b, 0.0)
        refs[-1][...] = x

    in_specs = [pl.BlockSpec((BM, Cin), lambda i: (i, 0))]
    args = [h]
    for (W, bvec) in wbs:
        in_specs.append(pl.BlockSpec(W.shape, lambda i: (0, 0)))
        in_specs.append(pl.BlockSpec((1, W.shape[1]), lambda i: (0, 0)))
        args += [W, bvec.reshape(1, -1)]
    out = pl.pallas_call(
        body,
        grid=(Mp // BM,),
        in_specs=in_specs,
        out_specs=pl.BlockSpec((BM, outC), lambda i: (i, 0)),
        out_shape=jax.ShapeDtypeStruct((Mp, outC), jnp.float32),
    )(*args)
    return out[:M]


_SC_NC = 2
_SC_NS = 16


def _sc_gather_rows(table, idx):
    V, D = table.shape
    M = idx.shape[0]
    NW = _SC_NC * _SC_NS
    mpw = M // NW
    ch = min(128, mpw, max(8, (32768 // D) // 8 * 8))
    while mpw % ch:
        ch -= 8
    nch = mpw // ch
    mesh = plsc.VectorSubcoreMesh(core_axis_name="c", subcore_axis_name="s")

    @functools.partial(
        pl.kernel, mesh=mesh,
        compiler_params=pltpu.CompilerParams(use_tc_tiling_on_sc=False),
        out_type=jax.ShapeDtypeStruct((M, D), jnp.float32),
        scratch_types=[
            pltpu.VMEM((mpw,), jnp.int32),
            pltpu.VMEM((ch, D), jnp.float32),
            pltpu.SemaphoreType.DMA,
        ],
    )
    def k(table_hbm, idx_hbm, out_hbm, idx_v, rows_v, sem):
        wid = lax.axis_index("s") * _SC_NC + lax.axis_index("c")
        base = wid * mpw
        pltpu.sync_copy(idx_hbm.at[pl.ds(base, mpw)], idx_v)

        def chunk(ci, _):
            off = ci * ch
            pltpu.async_copy(
                table_hbm.at[idx_v.at[pl.ds(off, ch)]], rows_v, sem).wait()
            pltpu.sync_copy(rows_v, out_hbm.at[pl.ds(base + off, ch)])
            return 0

        lax.fori_loop(0, nch, chunk, 0)

    return k(table, idx)


def _pad_lanes(x, mult=16):
    c = x.shape[-1]
    p = (-c) % mult
    if p:
        x = jnp.pad(x, [(0, 0)] * (x.ndim - 1) + [(0, p)])
    return x


def _pairwise_d2(a, b):
    aa = jnp.sum(a * a, axis=-1)[:, :, None]
    bb = jnp.sum(b * b, axis=-1)[:, None, :]
    ab = jnp.einsum('bnd,bmd->bnm', a, b)
    return jnp.maximum(aa + bb - 2.0 * ab, 0.0)


def _fps_new_xyz(xyz, npoint):
    B_, N, _ = xyz.shape
    Nr = N // 128
    planes = jnp.transpose(xyz, (2, 0, 1)).reshape(3, B_, Nr, 128)

    def body(p_ref, ox_ref, oy_ref, oz_ref):
        row_io = jax.lax.broadcasted_iota(jnp.int32, (Nr, 128), 0)
        col_io = jax.lax.broadcasted_iota(jnp.int32, (Nr, 128), 1)
        flat = row_io * 128 + col_io
        xs = [p_ref[0, b] for b in range(B_)]
        ys = [p_ref[1, b] for b in range(B_)]
        zs = [p_ref[2, b] for b in range(B_)]

        def step(i, carry):
            fars, dists = carry
            new_fars, new_dists = [], []
            for b in range(B_):
                x, y, z = xs[b], ys[b], zs[b]
                sel = flat == fars[b]
                cx = jnp.sum(jnp.where(sel, x, 0.0))
                cy = jnp.sum(jnp.where(sel, y, 0.0))
                cz = jnp.sum(jnp.where(sel, z, 0.0))
                ox_ref[pl.ds(i, 1), b] = jnp.full((1,), cx, jnp.float32)
                oy_ref[pl.ds(i, 1), b] = jnp.full((1,), cy, jnp.float32)
                oz_ref[pl.ds(i, 1), b] = jnp.full((1,), cz, jnp.float32)
                dx, dy, dz = x - cx, y - cy, z - cz
                d = dx * dx + dy * dy + dz * dz
                nd = jnp.minimum(dists[b], d)
                m = jnp.max(nd)
                nf = jnp.min(jnp.where(nd == m, flat, jnp.int32(N)))
                new_fars.append(nf)
                new_dists.append(nd)
            return (tuple(new_fars), tuple(new_dists))

        fars0 = tuple(jnp.int32(0) for _ in range(B_))
        dists0 = tuple(jnp.full((Nr, 128), 1e10, jnp.float32)
                       for _ in range(B_))
        jax.lax.fori_loop(0, npoint, step, (fars0, dists0))

    ox, oy, oz = pl.pallas_call(
        body,
        out_shape=[jax.ShapeDtypeStruct((npoint, B_), jnp.float32)] * 3,
    )(planes)
    return jnp.stack([ox, oy, oz], axis=-1).transpose(1, 0, 2)


def _pack_masks_pallas(new_xyz, xyz, r0, r1):
    B_, np_, _ = new_xyz.shape
    N = xyz.shape[1]
    W = N // 16
    CHN = min(2048, N)
    CW = CHN // 16
    nchunks = N // CHN
    BQ = min(256, np_)
    x2t = jnp.transpose(xyz, (0, 2, 1))
    jj = np.arange(CHN)
    pd_np = ((jj[:, None] // 16) == np.arange(CW)[None, :]).astype(
        np.float32) * (2.0 ** (jj % 16))[:, None]
    Pd = jnp.asarray(pd_np)
    r2_0 = np.float32(r0 * r0)
    r2_1 = np.float32(r1 * r1)

    def body(a_ref, bt_ref, pd_ref, o0_ref, o1_ref):
        a = a_ref[0]
        aa = jnp.sum(a * a, axis=1, keepdims=True)
        pd = pd_ref[...]
        for c in range(nchunks):
            btc = bt_ref[0][:, c * CHN:(c + 1) * CHN]
            bb = jnp.sum(btc * btc, axis=0, keepdims=True)
            ab = jnp.dot(a, btc, preferred_element_type=jnp.float32)
            d2 = jnp.maximum(aa + bb - 2.0 * ab, 0.0)
            m0 = (d2 < r2_0).astype(jnp.float32)
            m1 = (d2 < r2_1).astype(jnp.float32)
            p0 = jnp.dot(m0, pd, preferred_element_type=jnp.float32)
            p1 = jnp.dot(m1, pd, preferred_element_type=jnp.float32)
            o0_ref[0, :, c * CW:(c + 1) * CW] = p0.astype(jnp.int32)
            o1_ref[0, :, c * CW:(c + 1) * CW] = p1.astype(jnp.int32)

    return pl.pallas_call(
        body,
        grid=(B_, np_ // BQ),
        in_specs=[
            pl.BlockSpec((1, BQ, 3), lambda b, i: (b, i, 0)),
            pl.BlockSpec((1, 3, N), lambda b, i: (b, 0, 0)),
            pl.BlockSpec((CHN, CW), lambda b, i: (0, 0)),
        ],
        out_specs=[
            pl.BlockSpec((1, BQ, W), lambda b, i: (b, i, 0)),
            pl.BlockSpec((1, BQ, W), lambda b, i: (b, i, 0)),
        ],
        out_shape=[jax.ShapeDtypeStruct((B_, np_, W), jnp.int32)] * 2,
    )(new_xyz, x2t, Pd)


def _sc_ball_select(pk0, pk1, ns0, ns1):
    Q, W = pk1.shape
    NW = _SC_NC * _SC_NS
    G = Q // 16
    gpw = pl.cdiv(G, NW)
    mesh = plsc.VectorSubcoreMesh(core_axis_name="c", subcore_axis_name="s")

    @functools.partial(
        pl.kernel, mesh=mesh,
        compiler_params=pltpu.CompilerParams(use_tc_tiling_on_sc=False,
                                             needs_layout_passes=False),
        out_type=[jax.ShapeDtypeStruct((Q, ns0), jnp.int32),
                  jax.ShapeDtypeStruct((Q, ns1), jnp.int32)],
        scratch_types=[
            pltpu.VMEM((16, W), jnp.int32),
            pltpu.VMEM((16, W), jnp.int32),
            pltpu.VMEM((16, ns0), jnp.int32),
            pltpu.VMEM((16, ns1), jnp.int32),
        ],
    )
    def k(pk0_hbm, pk1_hbm, o0_hbm, o1_hbm, g0_v, g1_v, buf0, buf1):
        wid = lax.axis_index("s") * _SC_NC + lax.axis_index("c")
        lanes = jnp.arange(16, dtype=jnp.int32)
        ones = jnp.ones((16,), jnp.int32)

        for t in range(gpw):
            grp = wid + t * NW

            @pl.when(grp < G)
            def _():
                qbase = grp * 16
                pltpu.sync_copy(pk0_hbm.at[pl.ds(qbase, 16)], g0_v)
                pltpu.sync_copy(pk1_hbm.at[pl.ds(qbase, 16)], g1_v)

                def word(w, carry):
                    cnt0, cnt1 = carry
                    wsp = jnp.full((16,), w, jnp.int32)
                    wd0 = plsc.load_gather(g0_v, [lanes, wsp])
                    wd1 = plsc.load_gather(g1_v, [lanes, wsp])
                    done = (cnt0 >= ns0) & (cnt1 >= ns1)
                    wd1 = jnp.where(done, 0, wd1)

                    def peel_cond(st):
                        return jnp.sum(jnp.where(
                            st[0] != 0, ones, 0)) > 0

                    def peel(st):
                        w1, w0, c0, c1 = st
                        low = w1 & (-w1)
                        has = low != 0
                        bi = (plsc.bitcast(low.astype(jnp.float32),
                                           jnp.int32) >> 23) - 127
                        bi = jnp.where(has, bi, 0)
                        j = w * 16 + bi
                        m1 = has & (c1 < ns1)
                        plsc.store_scatter(buf1, [lanes, c1], j, mask=m1)
                        c1 = c1 + m1.astype(jnp.int32)
                        m0 = (((w0 >> bi) & 1) == 1) & has & (c0 < ns0)
                        plsc.store_scatter(buf0, [lanes, c0], j, mask=m0)
                        c0 = c0 + m0.astype(jnp.int32)
                        return (w1 - low, w0, c0, c1)

                    _, _, cnt0, cnt1 = lax.while_loop(
                        peel_cond, peel, (wd1, wd0, cnt0, cnt1))
                    return (cnt0, cnt1)

                z = jnp.zeros((16,), jnp.int32)
                cnt0, cnt1 = lax.fori_loop(0, W, word, (z, z))

                for buf, cnt, ns in ((buf0, cnt0, ns0), (buf1, cnt1, ns1)):
                    zsp = jnp.zeros((16,), jnp.int32)
                    first = plsc.load_gather(buf, [lanes, zsp])
                    first = jnp.where(cnt == 0, 0, first)
                    for s in range(ns):
                        val = plsc.load_gather(
                            buf, [lanes, jnp.full((16,), s, jnp.int32)])
                        val = jnp.where(s < cnt, val, first)
                        plsc.store_scatter(
                            buf, [lanes, jnp.full((16,), s, jnp.int32)],
                            val)
                pltpu.sync_copy(buf0, o0_hbm.at[pl.ds(qbase, 16)])
                pltpu.sync_copy(buf1, o1_hbm.at[pl.ds(qbase, 16)])

    return k(pk0, pk1)


def _batched_gather(x, idx):
    return jax.vmap(lambda a, i: a[i])(x, idx)


def _sa_msg(xyz, feats, npoint, radii, nsamples, scale_params):
    B_, N, _ = xyz.shape
    new_xyz = _fps_new_xyz(xyz, npoint)
    pk0, pk1 = _pack_masks_pallas(new_xyz, xyz, radii[0], radii[1])
    Q = B_ * npoint
    i0, i1 = _sc_ball_select(pk0.reshape(Q, -1), pk1.reshape(Q, -1),
                             nsamples[0], nsamples[1])
    idxs = [i0.reshape(B_, npoint, nsamples[0]),
            i1.reshape(B_, npoint, nsamples[1])]
    table = _pad_lanes(jnp.concatenate([xyz, feats], axis=-1)
                       if feats is not None else xyz)
    tflat = table.reshape(B_ * N, table.shape[-1])
    boff = (jnp.arange(B_, dtype=jnp.int32) * N)[:, None, None]
    outs = []
    for idx, nsample, params in zip(idxs, nsamples, scale_params):
        rows = _sc_gather_rows(tflat, (idx + boff).reshape(-1))
        rows = rows.reshape(B_, npoint, nsample, -1)
        grouped_xyz = rows[..., :3] - new_xyz[:, :, None, :]
        ci = 3 + (feats.shape[-1] if feats is not None else 0)
        g = jnp.concatenate([grouped_xyz, rows[..., 3:ci]], axis=-1)
        h = _mlp_pallas(g.reshape(B_ * npoint * nsample, ci),
                        _fold_bn(params))
        h = h.reshape(B_, npoint, nsample, -1)
        outs.append(jnp.max(h, axis=2))
    return new_xyz, jnp.concatenate(outs, axis=-1)


def _three_nn_pallas(xyz1, xyz2):
    B_, n1, _ = xyz1.shape
    n2 = xyz2.shape[1]
    BQ = min(512, n1)
    x2t = jnp.transpose(xyz2, (0, 2, 1))

    def body(a_ref, bt_ref, idx_ref, w_ref):
        a = a_ref[0]
        bt = bt_ref[0]
        aa = jnp.sum(a * a, axis=1, keepdims=True)
        bb = jnp.sum(bt * bt, axis=0, keepdims=True)
        ab = jnp.dot(a, bt, preferred_element_type=jnp.float32)
        d2 = jnp.maximum(aa + bb - 2.0 * ab, 0.0)
        col = jax.lax.broadcasted_iota(jnp.int32, (BQ, n2), 1)
        dks = []
        for k in range(3):
            m = jnp.min(d2, axis=1, keepdims=True)
            am = jnp.min(jnp.where(d2 == m, col, n2), axis=1, keepdims=True)
            idx_ref[0, :, k] = am[:, 0]
            dks.append(m)
            d2 = jnp.where(col == am, jnp.float32(jnp.inf), d2)
        dist = jnp.sqrt(jnp.concatenate(dks, axis=1))
        recip = 1.0 / (dist + 1e-8)
        w_ref[0] = recip / jnp.sum(recip, axis=1, keepdims=True)

    idx, w = pl.pallas_call(
        body,
        grid=(B_, n1 // BQ),
        in_specs=[
            pl.BlockSpec((1, BQ, 3), lambda b, i: (b, i, 0)),
            pl.BlockSpec((1, 3, n2), lambda b, i: (b, 0, 0)),
        ],
        out_specs=[
            pl.BlockSpec((1, BQ, 3), lambda b, i: (b, i, 0)),
            pl.BlockSpec((1, BQ, 3), lambda b, i: (b, i, 0)),
        ],
        out_shape=[
            jax.ShapeDtypeStruct((B_, n1, 3), jnp.int32),
            jax.ShapeDtypeStruct((B_, n1, 3), jnp.float32),
        ],
    )(xyz1, x2t)
    return idx, w


def _fp(xyz1, xyz2, feats1, feats2, params):
    B_, n1 = xyz1.shape[0], xyz1.shape[1]
    n2, C2 = feats2.shape[1], feats2.shape[2]
    idx, weight = _three_nn_pallas(xyz1, xyz2)
    boff = (jnp.arange(B_, dtype=jnp.int32) * n2)[:, None, None]
    rows = _sc_gather_rows(feats2.reshape(B_ * n2, C2),
                           (idx + boff).reshape(-1))
    gathered = rows.reshape(B_, n1, 3, C2)
    interp = jnp.sum(gathered * weight[..., None], axis=2)
    h = jnp.concatenate([interp, feats1], axis=-1) if feats1 is not None else interp
    bq, nq, ci = h.shape
    out = _mlp_pallas(h.reshape(bq * nq, ci), _fold_bn(params))
    return out.reshape(bq, nq, -1)


def kernel(points, params):
    xyz = points[:, :, :3]
    feats = points[:, :, 3:] if points.shape[2] > 3 else None
    l_xyz = [xyz]
    l_feats = [feats]
    for i in range(len(_NPOINTS)):
        nx, nf = _sa_msg(l_xyz[i], l_feats[i], _NPOINTS[i], _RADIUS[i],
                         _NSAMPLE[i], params["sa"][i])
        l_xyz.append(nx)
        l_feats.append(nf)
    for i in range(-1, -(len(_RADIUS) + 1), -1):
        l_feats[i - 1] = _fp(l_xyz[i - 1], l_xyz[i], l_feats[i - 1],
                             l_feats[i], params["fp"][i])
    return tuple(jnp.transpose(f, (0, 2, 1)) for f in l_feats)


# speedup vs baseline: 15.1308x; 1.0562x over previous
import functools

import jax
import jax.numpy as jnp
import numpy as np
from jax import lax
from jax.experimental import pallas as pl
from jax.experimental.pallas import tpu as pltpu
from jax.experimental.pallas import tpu_sc as plsc

_NPOINTS = [2048, 512, 128, 32]
_RADIUS = [[0.1, 0.5], [0.5, 1.0], [1.0, 2.0], [2.0, 4.0]]
_NSAMPLE = [[16, 32], [16, 32], [16, 32], [16, 32]]
_BN_EPS = 1e-5


def _fold_bn(params):
    out = []
    for (W, b, gamma, beta) in params:
        s = gamma / np.sqrt(1.0 + _BN_EPS)
        out.append((W * s[None, :], b * s + beta))
    return out


def _mlp_pallas(h, wbs):
    M, Cin = h.shape
    outC_all = max([Cin] + [w.shape[1] for (w, _) in wbs])
    BM = min(max(8, M), 4096,
             max(512, ((1 << 22) // (4 * outC_all)) // 256 * 256))
    Mp = pl.cdiv(M, BM) * BM
    if Mp != M:
        h = jnp.pad(h, ((0, Mp - M), (0, 0)))
    n = len(wbs)
    outC = wbs[-1][0].shape[1]

    def body(*refs):
        x = refs[0][...]
        for i in range(n):
            W = refs[1 + 2 * i][...]
            b = refs[2 + 2 * i][...]
            x = jnp.maximum(
                jnp.dot(x, W, preferred_element_type=jnp.float32) + b, 0.0)
        refs[-1][...] = x

    in_specs = [pl.BlockSpec((BM, Cin), lambda i: (i, 0))]
    args = [h]
    for (W, bvec) in wbs:
        in_specs.append(pl.BlockSpec(W.shape, lambda i: (0, 0)))
        in_specs.append(pl.BlockSpec((1, W.shape[1]), lambda i: (0, 0)))
        args += [W, bvec.reshape(1, -1)]
    out = pl.pallas_call(
        body,
        grid=(Mp // BM,),
        in_specs=in_specs,
        out_specs=pl.BlockSpec((BM, outC), lambda i: (i, 0)),
        out_shape=jax.ShapeDtypeStruct((Mp, outC), jnp.float32),
    )(*args)
    return out[:M]


_SC_NC = 2
_SC_NS = 16


def _sc_gather_rows(table, idx):
    V, D = table.shape
    M = idx.shape[0]
    NW = _SC_NC * _SC_NS
    mpw = M // NW
    ch = min(128, mpw, max(8, (32768 // D) // 8 * 8))
    while mpw % ch:
        ch -= 8
    nch = mpw // ch
    mesh = plsc.VectorSubcoreMesh(core_axis_name="c", subcore_axis_name="s")

    @functools.partial(
        pl.kernel, mesh=mesh,
        compiler_params=pltpu.CompilerParams(use_tc_tiling_on_sc=False),
        out_type=jax.ShapeDtypeStruct((M, D), jnp.float32),
        scratch_types=[
            pltpu.VMEM((mpw,), jnp.int32),
            pltpu.VMEM((ch, D), jnp.float32),
            pltpu.SemaphoreType.DMA,
        ],
    )
    def k(table_hbm, idx_hbm, out_hbm, idx_v, rows_v, sem):
        wid = lax.axis_index("s") * _SC_NC + lax.axis_index("c")
        base = wid * mpw
        pltpu.sync_copy(idx_hbm.at[pl.ds(base, mpw)], idx_v)

        def chunk(ci, _):
            off = ci * ch
            pltpu.async_copy(
                table_hbm.at[idx_v.at[pl.ds(off, ch)]], rows_v, sem).wait()
            pltpu.sync_copy(rows_v, out_hbm.at[pl.ds(base + off, ch)])
            return 0

        lax.fori_loop(0, nch, chunk, 0)

    return k(table, idx)


def _pad_lanes(x, mult=16):
    c = x.shape[-1]
    p = (-c) % mult
    if p:
        x = jnp.pad(x, [(0, 0)] * (x.ndim - 1) + [(0, p)])
    return x


def _pairwise_d2(a, b):
    aa = jnp.sum(a * a, axis=-1)[:, :, None]
    bb = jnp.sum(b * b, axis=-1)[:, None, :]
    ab = jnp.einsum('bnd,bmd->bnm', a, b)
    return jnp.maximum(aa + bb - 2.0 * ab, 0.0)


def _fps_new_xyz(xyz, npoint):
    B_, N, _ = xyz.shape
    Nr = N // 128
    planes = jnp.transpose(xyz, (2, 0, 1)).reshape(3, B_, Nr, 128)

    def body(p_ref, ox_ref, oy_ref, oz_ref):
        row_io = jax.lax.broadcasted_iota(jnp.int32, (Nr, 128), 0)
        col_io = jax.lax.broadcasted_iota(jnp.int32, (Nr, 128), 1)
        flat = row_io * 128 + col_io
        xs = [p_ref[0, b] for b in range(B_)]
        ys = [p_ref[1, b] for b in range(B_)]
        zs = [p_ref[2, b] for b in range(B_)]

        def step(i, carry):
            fars, dists = carry
            new_fars, new_dists = [], []
            for b in range(B_):
                x, y, z = xs[b], ys[b], zs[b]
                sel = flat == fars[b]
                cx = jnp.sum(jnp.where(sel, x, 0.0))
                cy = jnp.sum(jnp.where(sel, y, 0.0))
                cz = jnp.sum(jnp.where(sel, z, 0.0))
                ox_ref[pl.ds(i, 1), b] = jnp.full((1,), cx, jnp.float32)
                oy_ref[pl.ds(i, 1), b] = jnp.full((1,), cy, jnp.float32)
                oz_ref[pl.ds(i, 1), b] = jnp.full((1,), cz, jnp.float32)
                dx, dy, dz = x - cx, y - cy, z - cz
                d = dx * dx + dy * dy + dz * dz
                nd = jnp.minimum(dists[b], d)
                m = jnp.max(nd)
                nf = jnp.min(jnp.where(nd == m, flat, jnp.int32(N)))
                new_fars.append(nf)
                new_dists.append(nd)
            return (tuple(new_fars), tuple(new_dists))

        fars0 = tuple(jnp.int32(0) for _ in range(B_))
        dists0 = tuple(jnp.full((Nr, 128), 1e10, jnp.float32)
                       for _ in range(B_))
        jax.lax.fori_loop(0, npoint, step, (fars0, dists0))

    ox, oy, oz = pl.pallas_call(
        body,
        out_shape=[jax.ShapeDtypeStruct((npoint, B_), jnp.float32)] * 3,
    )(planes)
    return jnp.stack([ox, oy, oz], axis=-1).transpose(1, 0, 2)


def _pack_masks_pallas(new_xyz, xyz, r0, r1):
    B_, np_, _ = new_xyz.shape
    N = xyz.shape[1]
    W = N // 16
    CHN = min(2048, N)
    CW = CHN // 16
    nchunks = N // CHN
    BQ = min(256, np_)
    x2t = jnp.transpose(xyz, (0, 2, 1))
    jj = np.arange(CHN)
    pd_np = ((jj[:, None] // 16) == np.arange(CW)[None, :]).astype(
        np.float32) * (2.0 ** (jj % 16))[:, None]
    Pd = jnp.asarray(pd_np)
    r2_0 = np.float32(r0 * r0)
    r2_1 = np.float32(r1 * r1)

    def body(a_ref, bt_ref, pd_ref, o0_ref, o1_ref):
        a = a_ref[0]
        aa = jnp.sum(a * a, axis=1, keepdims=True)
        pd = pd_ref[...]
        for c in range(nchunks):
            btc = bt_ref[0][:, c * CHN:(c + 1) * CHN]
            bb = jnp.sum(btc * btc, axis=0, keepdims=True)
            ab = jnp.dot(a, btc, preferred_element_type=jnp.float32)
            d2 = jnp.maximum(aa + bb - 2.0 * ab, 0.0)
            m0 = (d2 < r2_0).astype(jnp.float32)
            m1 = (d2 < r2_1).astype(jnp.float32)
            p0 = jnp.dot(m0, pd, preferred_element_type=jnp.float32)
            p1 = jnp.dot(m1, pd, preferred_element_type=jnp.float32)
            o0_ref[0, :, c * CW:(c + 1) * CW] = p0.astype(jnp.int32)
            o1_ref[0, :, c * CW:(c + 1) * CW] = p1.astype(jnp.int32)

    return pl.pallas_call(
        body,
        grid=(B_, np_ // BQ),
        in_specs=[
            pl.BlockSpec((1, BQ, 3), lambda b, i: (b, i, 0)),
            pl.BlockSpec((1, 3, N), lambda b, i: (b, 0, 0)),
            pl.BlockSpec((CHN, CW), lambda b, i: (0, 0)),
        ],
        out_specs=[
            pl.BlockSpec((1, BQ, W), lambda b, i: (b, i, 0)),
            pl.BlockSpec((1, BQ, W), lambda b, i: (b, i, 0)),
        ],
        out_shape=[jax.ShapeDtypeStruct((B_, np_, W), jnp.int32)] * 2,
    )(new_xyz, x2t, Pd)


def _sc_ball_select(pk0, pk1, ns0, ns1):
    Q, W = pk1.shape
    NW = _SC_NC * _SC_NS
    G = Q // 16
    gpw = pl.cdiv(G, NW)
    mesh = plsc.VectorSubcoreMesh(core_axis_name="c", subcore_axis_name="s")

    @functools.partial(
        pl.kernel, mesh=mesh,
        compiler_params=pltpu.CompilerParams(use_tc_tiling_on_sc=False,
                                             needs_layout_passes=False),
        out_type=[jax.ShapeDtypeStruct((Q, ns0), jnp.int32),
                  jax.ShapeDtypeStruct((Q, ns1), jnp.int32)],
        scratch_types=[
            pltpu.VMEM((16, W), jnp.int32),
            pltpu.VMEM((16, W), jnp.int32),
            pltpu.VMEM((16, ns0), jnp.int32),
            pltpu.VMEM((16, ns1), jnp.int32),
        ],
    )
    def k(pk0_hbm, pk1_hbm, o0_hbm, o1_hbm, g0_v, g1_v, buf0, buf1):
        wid = lax.axis_index("s") * _SC_NC + lax.axis_index("c")
        lanes = jnp.arange(16, dtype=jnp.int32)
        ones = jnp.ones((16,), jnp.int32)

        for t in range(gpw):
            grp = wid + t * NW

            @pl.when(grp < G)
            def _():
                qbase = grp * 16
                pltpu.sync_copy(pk0_hbm.at[pl.ds(qbase, 16)], g0_v)
                pltpu.sync_copy(pk1_hbm.at[pl.ds(qbase, 16)], g1_v)

                def word(w, carry):
                    cnt0, cnt1 = carry
                    wsp = jnp.full((16,), w, jnp.int32)
                    wd0 = plsc.load_gather(g0_v, [lanes, wsp])
                    wd1 = plsc.load_gather(g1_v, [lanes, wsp])
                    done = (cnt0 >= ns0) & (cnt1 >= ns1)
                    wd1 = jnp.where(done, 0, wd1)

                    def peel_cond(st):
                        return jnp.sum(jnp.where(
                            st[0] != 0, ones, 0)) > 0

                    def peel(st):
                        w1, w0, c0, c1 = st
                        low = w1 & (-w1)
                        has = low != 0
                        bi = (plsc.bitcast(low.astype(jnp.float32),
                                           jnp.int32) >> 23) - 127
                        bi = jnp.where(has, bi, 0)
                        j = w * 16 + bi
                        m1 = has & (c1 < ns1)
                        plsc.store_scatter(buf1, [lanes, c1], j, mask=m1)
                        c1 = c1 + m1.astype(jnp.int32)
                        m0 = (((w0 >> bi) & 1) == 1) & has & (c0 < ns0)
                        plsc.store_scatter(buf0, [lanes, c0], j, mask=m0)
                        c0 = c0 + m0.astype(jnp.int32)
                        return (w1 - low, w0, c0, c1)

                    _, _, cnt0, cnt1 = lax.while_loop(
                        peel_cond, peel, (wd1, wd0, cnt0, cnt1))
                    return (cnt0, cnt1)

                z = jnp.zeros((16,), jnp.int32)
                cnt0, cnt1 = lax.fori_loop(0, W, word, (z, z))

                for buf, cnt, ns in ((buf0, cnt0, ns0), (buf1, cnt1, ns1)):
                    zsp = jnp.zeros((16,), jnp.int32)
                    first = plsc.load_gather(buf, [lanes, zsp])
                    first = jnp.where(cnt == 0, 0, first)
                    for s in range(ns):
                        val = plsc.load_gather(
                            buf, [lanes, jnp.full((16,), s, jnp.int32)])
                        val = jnp.where(s < cnt, val, first)
                        plsc.store_scatter(
                            buf, [lanes, jnp.full((16,), s, jnp.int32)],
                            val)
                pltpu.sync_copy(buf0, o0_hbm.at[pl.ds(qbase, 16)])
                pltpu.sync_copy(buf1, o1_hbm.at[pl.ds(qbase, 16)])

    return k(pk0, pk1)


def _batched_gather(x, idx):
    return jax.vmap(lambda a, i: a[i])(x, idx)


def _sa_msg(xyz, feats, npoint, radii, nsamples, scale_params):
    B_, N, _ = xyz.shape
    new_xyz = _fps_new_xyz(xyz, npoint)
    pk0, pk1 = _pack_masks_pallas(new_xyz, xyz, radii[0], radii[1])
    Q = B_ * npoint
    i0, i1 = _sc_ball_select(pk0.reshape(Q, -1), pk1.reshape(Q, -1),
                             nsamples[0], nsamples[1])
    idxs = [i0.reshape(B_, npoint, nsamples[0]),
            i1.reshape(B_, npoint, nsamples[1])]
    table = _pad_lanes(jnp.concatenate([xyz, feats], axis=-1)
                       if feats is not None else xyz)
    tflat = table.reshape(B_ * N, table.shape[-1])
    boff = (jnp.arange(B_, dtype=jnp.int32) * N)[:, None, None]
    outs = []
    for idx, nsample, params in zip(idxs, nsamples, scale_params):
        rows = _sc_gather_rows(tflat, (idx + boff).reshape(-1))
        rows = rows.reshape(B_, npoint, nsample, -1)
        grouped_xyz = rows[..., :3] - new_xyz[:, :, None, :]
        ci = 3 + (feats.shape[-1] if feats is not None else 0)
        g = jnp.concatenate([grouped_xyz, rows[..., 3:ci]], axis=-1)
        h = _mlp_pallas(g.reshape(B_ * npoint * nsample, ci),
                        _fold_bn(params))
        h = h.reshape(B_, npoint, nsample, -1)
        outs.append(jnp.max(h, axis=2))
    return new_xyz, jnp.concatenate(outs, axis=-1)


def _three_nn_pallas(xyz1, xyz2):
    B_, n1, _ = xyz1.shape
    n2 = xyz2.shape[1]
    BQ = min(512, n1)
    x2t = jnp.transpose(xyz2, (0, 2, 1))

    def body(a_ref, bt_ref, idx_ref, w_ref):
        a = a_ref[0]
        bt = bt_ref[0]
        aa = jnp.sum(a * a, axis=1, keepdims=True)
        bb = jnp.sum(bt * bt, axis=0, keepdims=True)
        ab = jnp.dot(a, bt, preferred_element_type=jnp.float32)
        d2 = jnp.maximum(aa + bb - 2.0 * ab, 0.0)
        col = jax.lax.broadcasted_iota(jnp.int32, (BQ, n2), 1)
        dks = []
        for k in range(3):
            m = jnp.min(d2, axis=1, keepdims=True)
            am = jnp.min(jnp.where(d2 == m, col, n2), axis=1, keepdims=True)
            idx_ref[0, :, k] = am[:, 0]
            dks.append(m)
            d2 = jnp.where(col == am, jnp.float32(jnp.inf), d2)
        dist = jnp.sqrt(jnp.concatenate(dks, axis=1))
        recip = 1.0 / (dist + 1e-8)
        w_ref[0] = recip / jnp.sum(recip, axis=1, keepdims=True)

    idx, w = pl.pallas_call(
        body,
        grid=(B_, n1 // BQ),
        in_specs=[
            pl.BlockSpec((1, BQ, 3), lambda b, i: (b, i, 0)),
            pl.BlockSpec((1, 3, n2), lambda b, i: (b, 0, 0)),
        ],
        out_specs=[
            pl.BlockSpec((1, BQ, 3), lambda b, i: (b, i, 0)),
            pl.BlockSpec((1, BQ, 3), lambda b, i: (b, i, 0)),
        ],
        out_shape=[
            jax.ShapeDtypeStruct((B_, n1, 3), jnp.int32),
            jax.ShapeDtypeStruct((B_, n1, 3), jnp.float32),
        ],
    )(xyz1, x2t)
    return idx, w


def _fp(xyz1, xyz2, feats1, feats2, params):
    B_, n1 = xyz1.shape[0], xyz1.shape[1]
    n2, C2 = feats2.shape[1], feats2.shape[2]
    idx, weight = _three_nn_pallas(xyz1, xyz2)
    boff = (jnp.arange(B_, dtype=jnp.int32) * n2)[:, None, None]
    rows = _sc_gather_rows(feats2.reshape(B_ * n2, C2),
                           (idx + boff).reshape(-1))
    gathered = rows.reshape(B_, n1, 3, C2)
    interp = jnp.sum(gathered * weight[..., None], axis=2)
    h = jnp.concatenate([interp, feats1], axis=-1) if feats1 is not None else interp
    bq, nq, ci = h.shape
    out = _mlp_pallas(h.reshape(bq * nq, ci), _fold_bn(params))
    return out.reshape(bq, nq, -1)


def kernel(points, params):
    xyz = points[:, :, :3]
    feats = points[:, :, 3:] if points.shape[2] > 3 else None
    l_xyz = [xyz]
    l_feats = [feats]
    for i in range(len(_NPOINTS)):
        nx, nf = _sa_msg(l_xyz[i], l_feats[i], _NPOINTS[i], _RADIUS[i],
                         _NSAMPLE[i], params["sa"][i])
        l_xyz.append(nx)
        l_feats.append(nf)
    for i in range(-1, -(len(_RADIUS) + 1), -1):
        l_feats[i - 1] = _fp(l_xyz[i - 1], l_xyz[i], l_feats[i - 1],
                             l_feats[i], params["fp"][i])
    return tuple(jnp.transpose(f, (0, 2, 1)) for f in l_feats)
